# baseline (speedup 1.0000x reference)
import numpy as np
import jax
import jax.numpy as jnp
from jax import lax
from jax.experimental import pallas as pl

_BSZ = 4096
_HYPO = 767
_DIM = 10
_SPAN = 1023

_ROT0 = (13, 15, 26, 6)
_ROT1 = (17, 29, 16, 24)


def _np_threefry2x32(k1, k2, x0, x1):
    k1 = np.uint32(k1)
    k2 = np.uint32(k2)
    x0 = np.asarray(x0, np.uint32).copy()
    x1 = np.asarray(x1, np.uint32).copy()
    ks = [k1, k2, np.uint32(k1 ^ k2 ^ np.uint32(0x1BD11BDA))]

    def rotl(v, r):
        return np.uint32((v << np.uint32(r)) | (v >> np.uint32(32 - r)))

    x0 += ks[0]
    x1 += ks[1]
    inject = [(1, 2, 1), (2, 0, 2), (0, 1, 3), (1, 2, 4), (2, 0, 5)]
    rots = [_ROT0, _ROT1, _ROT0, _ROT1, _ROT0]
    for (ia, ib, c), rr in zip(inject, rots):
        for r in rr:
            x0 = np.uint32(x0 + x1)
            x1 = rotl(x1, r)
            x1 = np.uint32(x0 ^ x1)
        x0 = np.uint32(x0 + ks[ia])
        x1 = np.uint32(x1 + ks[ib] + np.uint32(c))
    return x0, x1


_B1, _B2 = _np_threefry2x32(0, 42, np.array([0, 0]), np.array([0, 1]))
_K1A, _K1B = int(_B1[0]), int(_B2[0])
_K2A, _K2B = int(_B1[1]), int(_B2[1])


def _tf_bits(k1, k2, p):
    ks0 = jnp.uint32(k1)
    ks1 = jnp.uint32(k2)
    ks2 = jnp.uint32(k1 ^ k2 ^ 0x1BD11BDA)
    x0 = jnp.full(p.shape, ks0, jnp.uint32)
    x1 = p + ks1
    inject = [(ks1, ks2, 1), (ks2, ks0, 2), (ks0, ks1, 3),
              (ks1, ks2, 4), (ks2, ks0, 5)]
    rots = [_ROT0, _ROT1, _ROT0, _ROT1, _ROT0]
    for (ka, kb, c), rr in zip(inject, rots):
        for r in rr:
            x0 = x0 + x1
            x1 = (x1 << jnp.uint32(r)) | (x1 >> jnp.uint32(32 - r))
            x1 = x0 ^ x1
        x0 = x0 + ka
        x1 = x1 + (kb + jnp.uint32(c))
    return x0 ^ x1


def _mod1023(x):
    m = jnp.uint32(_SPAN)
    s = (x & m) + ((x >> jnp.uint32(10)) & m) + \
        ((x >> jnp.uint32(20)) & m) + (x >> jnp.uint32(30))
    s = (s & m) + (s >> jnp.uint32(10))
    return jnp.where(s >= m, s - m, s)


def _mod1023_small(x):
    m = jnp.uint32(_SPAN)
    s = (x & m) + (x >> jnp.uint32(10))
    return jnp.where(s >= m, s - m, s)


def _body(x_ref, out_ref, idx_ref):
    rows = idx_ref.shape[0]
    r = lax.broadcasted_iota(jnp.uint32, (rows, _HYPO), 0)
    h = lax.broadcasted_iota(jnp.uint32, (rows, _HYPO), 1)
    base = jnp.uint32(rows * _HYPO) * pl.program_id(0).astype(jnp.uint32)
    p = base + r * jnp.uint32(_HYPO) + h

    hi = _tf_bits(_K1A, _K1B, p)
    lo = _tf_bits(_K2A, _K2B, p)
    off = _mod1023_small(_mod1023(hi) * jnp.uint32(4) + _mod1023(lo))
    idx_ref[...] = off.astype(jnp.int32)

    g = (off + jnp.uint32(1)).astype(jnp.int32)
    for d in range(_DIM):
        mask = (g << (22 + d)) >> 31
        xd = x_ref[:, d].reshape(rows, 1).view(jnp.int32)
        out_ref[d] = (mask & xd).view(jnp.float32)


def kernel(x):
    rows = 16
    grid = _BSZ // rows
    out3, idx = pl.pallas_call(
        _body,
        grid=(grid,),
        in_specs=[pl.BlockSpec((rows, _DIM), lambda i: (i, 0))],
        out_specs=[
            pl.BlockSpec((_DIM, rows, _HYPO), lambda i: (0, i, 0)),
            pl.BlockSpec((rows, _HYPO), lambda i: (i, 0)),
        ],
        out_shape=[
            jax.ShapeDtypeStruct((_DIM, _BSZ, _HYPO), jnp.float32),
            jax.ShapeDtypeStruct((_BSZ, _HYPO), jnp.int32),
        ],
    )(x)
    return jnp.transpose(out3, (1, 2, 0)), idx


# speedup vs baseline: 1.1057x; 1.1057x over previous
import numpy as np
import jax
import jax.numpy as jnp
from jax import lax
from jax.experimental import pallas as pl

_BSZ = 4096
_HYPO = 767
_DIM = 10
_SPAN = 1023

_ROT0 = (13, 15, 26, 6)
_ROT1 = (17, 29, 16, 24)


def _np_threefry2x32(k1, k2, x0, x1):
    k1 = np.uint32(k1)
    k2 = np.uint32(k2)
    x0 = np.asarray(x0, np.uint32).copy()
    x1 = np.asarray(x1, np.uint32).copy()
    ks = [k1, k2, np.uint32(k1 ^ k2 ^ np.uint32(0x1BD11BDA))]

    def rotl(v, r):
        return np.uint32((v << np.uint32(r)) | (v >> np.uint32(32 - r)))

    x0 += ks[0]
    x1 += ks[1]
    inject = [(1, 2, 1), (2, 0, 2), (0, 1, 3), (1, 2, 4), (2, 0, 5)]
    rots = [_ROT0, _ROT1, _ROT0, _ROT1, _ROT0]
    for (ia, ib, c), rr in zip(inject, rots):
        for r in rr:
            x0 = np.uint32(x0 + x1)
            x1 = rotl(x1, r)
            x1 = np.uint32(x0 ^ x1)
        x0 = np.uint32(x0 + ks[ia])
        x1 = np.uint32(x1 + ks[ib] + np.uint32(c))
    return x0, x1


_B1, _B2 = _np_threefry2x32(0, 42, np.array([0, 0]), np.array([0, 1]))
_K1A, _K1B = int(_B1[0]), int(_B2[0])
_K2A, _K2B = int(_B1[1]), int(_B2[1])


def _tf_bits(k1, k2, p):
    ks0 = jnp.uint32(k1)
    ks1 = jnp.uint32(k2)
    ks2 = jnp.uint32(k1 ^ k2 ^ 0x1BD11BDA)
    x0 = jnp.full(p.shape, ks0, jnp.uint32)
    x1 = p + ks1
    inject = [(ks1, ks2, 1), (ks2, ks0, 2), (ks0, ks1, 3),
              (ks1, ks2, 4), (ks2, ks0, 5)]
    rots = [_ROT0, _ROT1, _ROT0, _ROT1, _ROT0]
    for (ka, kb, c), rr in zip(inject, rots):
        for r in rr:
            x0 = x0 + x1
            x1 = (x1 << jnp.uint32(r)) | (x1 >> jnp.uint32(32 - r))
            x1 = x0 ^ x1
        x0 = x0 + ka
        x1 = x1 + (kb + jnp.uint32(c))
    return x0 ^ x1


def _mod1023(x):
    m = jnp.uint32(_SPAN)
    s = (x & m) + ((x >> jnp.uint32(10)) & m) + \
        ((x >> jnp.uint32(20)) & m) + (x >> jnp.uint32(30))
    s = (s & m) + (s >> jnp.uint32(10))
    return jnp.where(s >= m, s - m, s)


def _mod1023_small(x):
    m = jnp.uint32(_SPAN)
    s = (x & m) + (x >> jnp.uint32(10))
    return jnp.where(s >= m, s - m, s)


def _body(x_ref, out_ref, idx_ref):
    rows = idx_ref.shape[0]
    r = lax.broadcasted_iota(jnp.uint32, (rows, _HYPO), 0)
    h = lax.broadcasted_iota(jnp.uint32, (rows, _HYPO), 1)
    base = jnp.uint32(rows * _HYPO) * pl.program_id(0).astype(jnp.uint32)
    p = base + r * jnp.uint32(_HYPO) + h

    hi = _tf_bits(_K1A, _K1B, p)
    lo = _tf_bits(_K2A, _K2B, p)
    off = _mod1023_small(_mod1023(hi) * jnp.uint32(4) + _mod1023(lo))
    idx_ref[...] = off.astype(jnp.int32)

    g = (off + jnp.uint32(1)).astype(jnp.int32)
    for d in range(_DIM):
        mask = (g << (22 + d)) >> 31
        xd = x_ref[:, d].reshape(rows, 1).view(jnp.int32)
        out_ref[d] = (mask & xd).view(jnp.float32)


def kernel(x):
    rows = 64
    grid = _BSZ // rows
    out3, idx = pl.pallas_call(
        _body,
        grid=(grid,),
        in_specs=[pl.BlockSpec((rows, _DIM), lambda i: (i, 0))],
        out_specs=[
            pl.BlockSpec((_DIM, rows, _HYPO), lambda i: (0, i, 0)),
            pl.BlockSpec((rows, _HYPO), lambda i: (i, 0)),
        ],
        out_shape=[
            jax.ShapeDtypeStruct((_DIM, _BSZ, _HYPO), jnp.float32),
            jax.ShapeDtypeStruct((_BSZ, _HYPO), jnp.int32),
        ],
    )(x)
    return jnp.transpose(out3, (1, 2, 0)), idx


# speedup vs baseline: 1.4293x; 1.2927x over previous
import numpy as np
import jax
import jax.numpy as jnp
from jax import lax
from jax.experimental import pallas as pl

_BSZ = 4096
_HYPO = 767
_DIM = 10
_SPAN = 1023

_ROT0 = (13, 15, 26, 6)
_ROT1 = (17, 29, 16, 24)


def _np_threefry2x32(k1, k2, x0, x1):
    k1 = np.uint32(k1)
    k2 = np.uint32(k2)
    x0 = np.asarray(x0, np.uint32).copy()
    x1 = np.asarray(x1, np.uint32).copy()
    ks = [k1, k2, np.uint32(k1 ^ k2 ^ np.uint32(0x1BD11BDA))]

    def rotl(v, r):
        return np.uint32((v << np.uint32(r)) | (v >> np.uint32(32 - r)))

    x0 += ks[0]
    x1 += ks[1]
    inject = [(1, 2, 1), (2, 0, 2), (0, 1, 3), (1, 2, 4), (2, 0, 5)]
    rots = [_ROT0, _ROT1, _ROT0, _ROT1, _ROT0]
    for (ia, ib, c), rr in zip(inject, rots):
        for r in rr:
            x0 = np.uint32(x0 + x1)
            x1 = rotl(x1, r)
            x1 = np.uint32(x0 ^ x1)
        x0 = np.uint32(x0 + ks[ia])
        x1 = np.uint32(x1 + ks[ib] + np.uint32(c))
    return x0, x1


_B1, _B2 = _np_threefry2x32(0, 42, np.array([0, 0]), np.array([0, 1]))
_K1A, _K1B = int(_B1[0]), int(_B2[0])
_K2A, _K2B = int(_B1[1]), int(_B2[1])


def _tf_bits(k1, k2, x1):
    ks0 = jnp.uint32(k1)
    ks1 = jnp.uint32(k2)
    ks2 = jnp.uint32(k1 ^ k2 ^ 0x1BD11BDA)
    inject = [(ks1, ks2, 1), (ks2, ks0, 2), (ks0, ks1, 3),
              (ks1, ks2, 4), (ks2, ks0, 5)]
    rots = [_ROT0, _ROT1, _ROT0, _ROT1, _ROT0]
    x0 = None
    for (ka, kb, c), rr in zip(inject, rots):
        for r in rr:
            x0 = (x1 + ks0) if x0 is None else (x0 + x1)
            x1 = (x1 << jnp.uint32(r)) | (x1 >> jnp.uint32(32 - r))
            x1 = x0 ^ x1
        x0 = x0 + ka
        x1 = x1 + (kb + jnp.uint32(c))
    return x0 ^ x1


def _modsum1023(x):
    m = jnp.uint32(_SPAN)
    s = (x & m) + ((x >> jnp.uint32(10)) & m) + \
        ((x >> jnp.uint32(20)) & m) + (x >> jnp.uint32(30))
    return (s & m) + (s >> jnp.uint32(10))


def _body(x_ref, pa_ref, pb_ref, out_ref, idx_ref):
    rows = idx_ref.shape[0]
    base = jnp.uint32(rows * _HYPO) * pl.program_id(0).astype(jnp.uint32)

    hi = _tf_bits(_K1A, _K1B, pa_ref[...] + base)
    lo = _tf_bits(_K2A, _K2B, pb_ref[...] + base)
    acc = (_modsum1023(hi) << jnp.uint32(2)) + _modsum1023(lo)
    s = (acc & jnp.uint32(_SPAN)) + (acc >> jnp.uint32(10))
    off = jnp.where(s >= jnp.uint32(_SPAN), s - jnp.uint32(_SPAN), s)
    idx_ref[...] = off.view(jnp.int32)

    g = off.view(jnp.int32) + 1
    for d in range(_DIM):
        mask = (g << (22 + d)) >> 31
        xd = x_ref[:, d].reshape(rows, 1).view(jnp.int32)
        out_ref[d] = (mask & xd).view(jnp.float32)


def kernel(x):
    rows = 32
    grid = _BSZ // rows
    p0 = (np.arange(rows, dtype=np.uint32)[:, None] * np.uint32(_HYPO)
          + np.arange(_HYPO, dtype=np.uint32)[None, :])
    pa = jnp.asarray(p0 + np.uint32(_K1B))
    pb = jnp.asarray(p0 + np.uint32(_K2B))
    out3, idx = pl.pallas_call(
        _body,
        grid=(grid,),
        in_specs=[
            pl.BlockSpec((rows, _DIM), lambda i: (i, 0)),
            pl.BlockSpec((rows, _HYPO), lambda i: (0, 0)),
            pl.BlockSpec((rows, _HYPO), lambda i: (0, 0)),
        ],
        out_specs=[
            pl.BlockSpec((_DIM, rows, _HYPO), lambda i: (0, i, 0)),
            pl.BlockSpec((rows, _HYPO), lambda i: (i, 0)),
        ],
        out_shape=[
            jax.ShapeDtypeStruct((_DIM, _BSZ, _HYPO), jnp.float32),
            jax.ShapeDtypeStruct((_BSZ, _HYPO), jnp.int32),
        ],
    )(x, pa, pb)
    return jnp.transpose(out3, (1, 2, 0)), idx
